# baseline (speedup 1.0000x reference)
import numpy as np
import jax
import jax.numpy as jnp
from jax import lax
from jax.experimental import pallas as pl
from jax.experimental.pallas import tpu as pltpu
from jax.experimental.pallas import tpu_sc as plsc

_N = 10000
_E = 320000
_F = 128
_HID = 16
_HEADS = 8
_HH = _HEADS * _HID

_B = 128
_NC, _NS = 2, 16
_NW = _NC * _NS
_CHUNKS = _E // _B
_CH_BASE = _CHUNKS // _NW
_CH_REM = _CHUNKS - _CH_BASE * _NW
_NP = 10240
_RPT = _NP // _NS
_ZR = _B
_NZ = _RPT // _ZR
_AW = _HH + 16

_IPERM = np.empty(_HH, dtype=np.int32)
for _k in range(_HID):
    for _hd in range(_HEADS):
        _IPERM[_k * _HEADS + _hd] = _hd * _HID + _k

_TAU = np.empty(_HH, dtype=np.int32)
for _t in range(4):
    for _i in range(16):
        _TAU[32 * _t + 2 * _i] = 32 * _t + _i
        _TAU[32 * _t + 2 * _i + 1] = 32 * _t + 16 + _i

_MEAN = np.zeros((_HH, _HID), dtype=np.float32)
_MEAN[np.arange(_HH), np.arange(_HH) // _HEADS] = 1.0 / _HEADS

_BLK = 2048
_FBLK = 2000


def _d1_body(x_ref, w_ref, wt_ref, a_ref, hb_ref, ao_ref):
    x = x_ref[...]
    h = jnp.dot(x, w_ref[...], preferred_element_type=jnp.float32)
    hb_ref[...] = jnp.dot(
        x, wt_ref[...], preferred_element_type=jnp.float32
    ).astype(jnp.bfloat16)
    ao_ref[...] = jnp.dot(h, a_ref[...], preferred_element_type=jnp.float32)


def _dense1(x, W1p, W1pt, A1):
    return pl.pallas_call(
        _d1_body,
        grid=(_NP // _BLK,),
        in_specs=[
            pl.BlockSpec((_BLK, _F), lambda i: (i, 0)),
            pl.BlockSpec((_F, _HH), lambda i: (0, 0)),
            pl.BlockSpec((_F, _HH), lambda i: (0, 0)),
            pl.BlockSpec((_HH, 32), lambda i: (0, 0)),
        ],
        out_specs=[
            pl.BlockSpec((_BLK, _HH), lambda i: (i, 0)),
            pl.BlockSpec((_BLK, 32), lambda i: (i, 0)),
        ],
        out_shape=[
            jax.ShapeDtypeStruct((_NP, _HH), jnp.bfloat16),
            jax.ShapeDtypeStruct((_NP, 32), jnp.float32),
        ],
    )(x, W1p, W1pt, A1)


def _mid_body(acc_ref, b_ref, w_ref, wt_ref, a2_ref, hb_ref, ao_ref):
    acc = acc_ref[0] + acc_ref[1]
    o = acc[:, :_HH]
    d16 = acc[:, _HH:]
    dg = jnp.tile(d16, (1, _HEADS))
    h1 = jnp.maximum(o / (dg + 1e-16) + b_ref[...], 0.0)
    h2 = jnp.dot(h1, w_ref[...], preferred_element_type=jnp.float32)
    hb_ref[...] = jnp.dot(
        h1, wt_ref[...], preferred_element_type=jnp.float32
    ).astype(jnp.bfloat16)
    ao_ref[...] = jnp.dot(h2, a2_ref[...], preferred_element_type=jnp.float32)


def _dense2(acc1, b1p, W2pp, W2ppt, A2):
    return pl.pallas_call(
        _mid_body,
        grid=(_NP // _BLK,),
        in_specs=[
            pl.BlockSpec((_NC, _BLK, _AW), lambda i: (0, i, 0)),
            pl.BlockSpec((1, _HH), lambda i: (0, 0)),
            pl.BlockSpec((_HH, _HH), lambda i: (0, 0)),
            pl.BlockSpec((_HH, _HH), lambda i: (0, 0)),
            pl.BlockSpec((_HH, 32), lambda i: (0, 0)),
        ],
        out_specs=[
            pl.BlockSpec((_BLK, _HH), lambda i: (i, 0)),
            pl.BlockSpec((_BLK, 32), lambda i: (i, 0)),
        ],
        out_shape=[
            jax.ShapeDtypeStruct((_NP, _HH), jnp.bfloat16),
            jax.ShapeDtypeStruct((_NP, 32), jnp.float32),
        ],
    )(acc1, b1p, W2pp, W2ppt, A2)


def _fin_body(acc_ref, b_ref, m_ref, out_ref):
    acc = acc_ref[0] + acc_ref[1]
    o = acc[:, :_HH]
    d16 = acc[:, _HH:]
    dg = jnp.tile(d16, (1, _HEADS))
    on = o / (dg + 1e-16)
    out_ref[...] = (
        jnp.dot(on, m_ref[...], preferred_element_type=jnp.float32) + b_ref[...]
    )


def _final(acc2, b2, M):
    return pl.pallas_call(
        _fin_body,
        grid=(_N // _FBLK,),
        in_specs=[
            pl.BlockSpec((_NC, _FBLK, _AW), lambda i: (0, i, 0)),
            pl.BlockSpec((1, _HID), lambda i: (0, 0)),
            pl.BlockSpec((_HH, _HID), lambda i: (0, 0)),
        ],
        out_specs=pl.BlockSpec((_FBLK, _HID), lambda i: (i, 0)),
        out_shape=jax.ShapeDtypeStruct((_N, _HID), jnp.float32),
    )(acc2, b2, M)


_HIMASK = np.int32(-65536)


def _edge_body(src_hbm, dst_hbm, comb_hbm, ad_hbm, out_hbm,
               idx_s, idx_d, cb, eb, msg, sem, acc):
    cid = lax.axis_index("c")
    sid = lax.axis_index("s")
    wid = sid * _NC + cid

    def zrow(j, carry):
        for t in range(_AW // 16):
            msg[j, pl.ds(t * 16, 16)] = jnp.zeros((16,), jnp.float32)
        return carry

    lax.fori_loop(0, _B, zrow, 0)
    row0 = sid * _RPT
    for z in range(_NZ):
        pltpu.sync_copy(msg.at[pl.ds(0, _ZR)],
                        acc.at[pl.ds(row0 + z * _ZR, _ZR)])
    plsc.subcore_barrier()

    n_my = jnp.where(wid < _CH_REM, _CH_BASE + 1, _CH_BASE)

    def chunk(k, carry):
        base = (k * _NW + wid) * _B
        pltpu.sync_copy(src_hbm.at[pl.ds(base, _B)], idx_s)
        pltpu.sync_copy(dst_hbm.at[pl.ds(base, _B)], idx_d)
        cp_c = pltpu.async_copy(comb_hbm.at[idx_s], cb, sem)
        cp_b = pltpu.async_copy(ad_hbm.at[idx_d], eb, sem)
        cp_c.wait()
        cp_b.wait()

        def edge(j, c2):
            sa = plsc.bitcast(cb[j, pl.ds(64, 16)], jnp.float32)
            e = sa + eb[j, :]
            e = jnp.maximum(e, 0.2 * e)
            ev = jnp.exp(e)
            msg[j, pl.ds(_HH, 16)] = ev
            for t in range(4):
                vi = cb[j, pl.ds(16 * t, 16)]
                lo = plsc.bitcast(lax.shift_left(vi, 16), jnp.float32)
                hi = plsc.bitcast(lax.bitwise_and(vi, _HIMASK), jnp.float32)
                msg[j, pl.ds(32 * t, 16)] = lo * ev
                msg[j, pl.ds(32 * t + 16, 16)] = hi * ev
            return c2

        lax.fori_loop(0, _B, edge, 0, unroll=2)
        pltpu.sync_copy(msg, acc.at[idx_d], add=True)
        return carry

    lax.fori_loop(0, n_my, chunk, 0)
    plsc.subcore_barrier()

    for z in range(_NZ):
        r = row0 + z * _ZR
        pltpu.sync_copy(acc.at[pl.ds(r, _ZR)], out_hbm.at[cid, pl.ds(r, _ZR)])


def _edge_call(src, dst, comb, a_dst):
    mesh = plsc.VectorSubcoreMesh(core_axis_name="c", subcore_axis_name="s",
                                  num_cores=_NC, num_subcores=_NS)
    return pl.kernel(
        _edge_body,
        out_type=jax.ShapeDtypeStruct((_NC, _NP, _AW), jnp.float32),
        mesh=mesh,
        scratch_types=[
            pltpu.VMEM((_B,), jnp.int32),
            pltpu.VMEM((_B,), jnp.int32),
            pltpu.VMEM((_B, 80), jnp.int32),
            pltpu.VMEM((_B, 16), jnp.float32),
            pltpu.VMEM((_B, _AW), jnp.float32),
            pltpu.SemaphoreType.DMA,
            pltpu.VMEM_SHARED((_NP, _AW), jnp.float32),
        ],
        compiler_params=pltpu.CompilerParams(use_tc_tiling_on_sc=False,
                                            needs_layout_passes=False),
    )(src, dst, comb, a_dst)


def _build_A(att_s, att_d, perm):
    rows = jnp.arange(_HH, dtype=jnp.int32)
    cols = rows // _HID
    Bs = jnp.zeros((_HH, _HEADS), jnp.float32).at[rows, cols].set(
        att_s.reshape(-1))[perm]
    Bd = jnp.zeros((_HH, _HEADS), jnp.float32).at[rows, cols].set(
        att_d.reshape(-1))[perm]
    return jnp.concatenate([Bs, Bs, Bd, Bd], axis=1)


def _pack_comb(hb, a):
    hi = lax.bitcast_convert_type(hb.reshape(_NP, 64, 2), jnp.int32)
    ai = lax.bitcast_convert_type(a[:, :16], jnp.int32)
    return jnp.concatenate([hi, ai], axis=1)


def kernel(x, edge_index, W1, att_src1, att_dst1, b1,
           W2, att_src2, att_dst2, b2):
    perm = jnp.asarray(_IPERM)
    tau = jnp.asarray(_TAU)
    W1p = W1[:, perm]
    W1pt = W1p[:, tau]
    W2pp = W2[perm][:, perm]
    W2ppt = W2pp[:, tau]
    b1p = b1[perm].reshape(1, _HH)
    A1 = _build_A(att_src1, att_dst1, perm)
    A2 = _build_A(att_src2, att_dst2, perm)
    src = edge_index[0]
    dst = edge_index[1]
    xp = jnp.pad(x, ((0, _NP - _N), (0, 0)))

    hb1, a1 = _dense1(xp, W1p, W1pt, A1)
    acc1 = _edge_call(src, dst, _pack_comb(hb1, a1), a1[:, 16:])
    hb2, a2 = _dense2(acc1, b1p, W2pp, W2ppt, A2)
    acc2 = _edge_call(src, dst, _pack_comb(hb2, a2), a2[:, 16:])
    return _final(acc2, b2.reshape(1, _HID), jnp.asarray(_MEAN))


# speedup vs baseline: 3.7095x; 3.7095x over previous
import numpy as np
import jax
import jax.numpy as jnp
from jax import lax
from jax.experimental import pallas as pl
from jax.experimental.pallas import tpu as pltpu
from jax.experimental.pallas import tpu_sc as plsc

_N = 10000
_E = 320000
_F = 128
_HID = 16
_HEADS = 8
_HH = _HEADS * _HID

_B = 64
_NC, _NS = 2, 16
_NW = _NC * _NS
_CHUNKS = _E // _B
_CHT = _CHUNKS // _NW
_CH_REM = _CHUNKS - _CHT * _NW
_NP = 10112
_RPT = _NP // _NS
_AW = _HH + 16

_IPERM = np.empty(_HH, dtype=np.int32)
for _k in range(_HID):
    for _hd in range(_HEADS):
        _IPERM[_k * _HEADS + _hd] = _hd * _HID + _k

_TAU = np.empty(_HH, dtype=np.int32)
for _t in range(4):
    for _i in range(16):
        _TAU[32 * _t + 2 * _i] = 32 * _t + _i
        _TAU[32 * _t + 2 * _i + 1] = 32 * _t + 16 + _i

_MEAN = np.zeros((_HH, _HID), dtype=np.float32)
_MEAN[np.arange(_HH), np.arange(_HH) // _HEADS] = 1.0 / _HEADS

_BLK = 1264
_FBLK = 2000


def _pack_comb_tc(y, comb_ref):
    he = y[:, :64].astype(jnp.bfloat16).astype(jnp.float32)
    ho = y[:, 64:128].astype(jnp.bfloat16).astype(jnp.float32)
    ie = lax.shift_right_logical(lax.bitcast_convert_type(he, jnp.int32), 16)
    io = lax.bitwise_and(lax.bitcast_convert_type(ho, jnp.int32), _HIMASK)
    comb_ref[:, :64] = lax.bitwise_or(io, ie)
    comb_ref[:, 64:] = lax.bitcast_convert_type(y[:, 128:144], jnp.int32)


def _d1_body(x_ref, w_ref, comb_ref, ad_ref):
    y = jnp.dot(x_ref[...], w_ref[...], preferred_element_type=jnp.float32)
    _pack_comb_tc(y, comb_ref)
    ad_ref[...] = y[:, 144:]


def _dense1(x, WW1):
    return pl.pallas_call(
        _d1_body,
        grid=(_NP // _BLK,),
        in_specs=[
            pl.BlockSpec((_BLK, _F), lambda i: (i, 0)),
            pl.BlockSpec((_F, 160), lambda i: (0, 0)),
        ],
        out_specs=[
            pl.BlockSpec((_BLK, 80), lambda i: (i, 0)),
            pl.BlockSpec((_BLK, 16), lambda i: (i, 0)),
        ],
        out_shape=[
            jax.ShapeDtypeStruct((_NP, 80), jnp.int32),
            jax.ShapeDtypeStruct((_NP, 16), jnp.float32),
        ],
    )(x, WW1)


def _mid_body(acc_ref, b_ref, w_ref, comb_ref, ad_ref):
    acc = acc_ref[0] + acc_ref[1]
    o = acc[:, :_HH]
    d16 = acc[:, _HH:]
    dg = jnp.tile(d16, (1, _HEADS))
    h1 = jnp.maximum(o / (dg + 1e-16) + b_ref[...], 0.0)
    y = jnp.dot(h1, w_ref[...], preferred_element_type=jnp.float32)
    _pack_comb_tc(y, comb_ref)
    ad_ref[...] = y[:, 144:]


def _dense2(acc1, b1p, WW2):
    return pl.pallas_call(
        _mid_body,
        grid=(_NP // _BLK,),
        in_specs=[
            pl.BlockSpec((_NC, _BLK, _AW), lambda i: (0, i, 0)),
            pl.BlockSpec((1, _HH), lambda i: (0, 0)),
            pl.BlockSpec((_HH, 160), lambda i: (0, 0)),
        ],
        out_specs=[
            pl.BlockSpec((_BLK, 80), lambda i: (i, 0)),
            pl.BlockSpec((_BLK, 16), lambda i: (i, 0)),
        ],
        out_shape=[
            jax.ShapeDtypeStruct((_NP, 80), jnp.int32),
            jax.ShapeDtypeStruct((_NP, 16), jnp.float32),
        ],
    )(acc1, b1p, WW2)


def _fin_body(acc_ref, b_ref, m_ref, out_ref):
    acc = acc_ref[0] + acc_ref[1]
    o = acc[:, :_HH]
    d16 = acc[:, _HH:]
    dg = jnp.tile(d16, (1, _HEADS))
    on = o / (dg + 1e-16)
    out_ref[...] = (
        jnp.dot(on, m_ref[...], preferred_element_type=jnp.float32) + b_ref[...]
    )


def _final(acc2, b2, M):
    return pl.pallas_call(
        _fin_body,
        grid=(_N // _FBLK,),
        in_specs=[
            pl.BlockSpec((_NC, _FBLK, _AW), lambda i: (0, i, 0)),
            pl.BlockSpec((1, _HID), lambda i: (0, 0)),
            pl.BlockSpec((_HH, _HID), lambda i: (0, 0)),
        ],
        out_specs=pl.BlockSpec((_FBLK, _HID), lambda i: (i, 0)),
        out_shape=jax.ShapeDtypeStruct((_N, _HID), jnp.float32),
    )(acc2, b2, M)


_HIMASK = np.int32(-65536)


def _edge_compute(cbX, ebX, msgX):
    @plsc.parallel_loop(0, _B, unroll=4)
    def _edge(j):
        sa = plsc.bitcast(cbX[j, pl.ds(64, 16)], jnp.float32)
        e = sa + ebX[j, :]
        e = jnp.maximum(e, 0.2 * e)
        ev = jnp.exp(e)
        msgX[j, pl.ds(_HH, 16)] = ev
        for t in range(4):
            vi = cbX[j, pl.ds(16 * t, 16)]
            lo = plsc.bitcast(lax.shift_left(vi, 16), jnp.float32)
            hi = plsc.bitcast(lax.bitwise_and(vi, _HIMASK), jnp.float32)
            msgX[j, pl.ds(32 * t, 16)] = lo * ev
            msgX[j, pl.ds(32 * t + 16, 16)] = hi * ev


def _edge_body(src_hbm, dst_hbm, comb_hbm, ad_hbm, out_hbm,
               cb0, eb0, is0, id0, cb1, eb1, is1, id1, cb2, eb2, is2, id2,
               msg0, sx0, msg1, sx1,
               sg0, sg1, sg2, si0, si1, si2, ss0, ss1, acc):
    cid = lax.axis_index("c")
    sid = lax.axis_index("s")
    wid = sid * _NC + cid
    gsets = ((cb0, eb0, is0, id0, sg0, si0),
             (cb1, eb1, is1, id1, sg1, si1),
             (cb2, eb2, is2, id2, sg2, si2))
    ssets = ((msg0, sx0, ss0), (msg1, sx1, ss1))

    @plsc.parallel_loop(0, _B, unroll=4)
    def _zrow(j):
        for t in range(_AW // 16):
            msg0[j, pl.ds(t * 16, 16)] = jnp.zeros((16,), jnp.float32)

    row0 = sid * _RPT
    for z in range(10):
        r0, rows = (64 * z, 64) if z < 9 else (576, 56)
        pltpu.sync_copy(msg0.at[pl.ds(0, rows)],
                        acc.at[pl.ds(row0 + r0, rows)])
    plsc.subcore_barrier()

    def idx_load(k, iS, iD, sem):
        base = (k * _NW + wid) * _B
        ca = pltpu.async_copy(src_hbm.at[pl.ds(base, _B)], iS, sem)
        cb_ = pltpu.async_copy(dst_hbm.at[pl.ds(base, _B)], iD, sem)
        return ca, cb_

    def idx_wait(k, iS, iD, sem):
        base = (k * _NW + wid) * _B
        pltpu.make_async_copy(src_hbm.at[pl.ds(base, _B)], iS, sem).wait()
        pltpu.make_async_copy(dst_hbm.at[pl.ds(base, _B)], iD, sem).wait()

    def gather_issue(cbX, ebX, iS, iD, sem):
        pltpu.async_copy(comb_hbm.at[iS], cbX, sem)
        pltpu.async_copy(ad_hbm.at[iD], ebX, sem)

    def gather_wait(cbX, ebX, iS, iD, sem):
        pltpu.make_async_copy(comb_hbm.at[iS], cbX, sem).wait()
        pltpu.make_async_copy(ad_hbm.at[iD], ebX, sem).wait()

    @pl.when(wid < _CH_REM)
    def _():
        base = (_CHT * _NW + wid) * _B
        ca = pltpu.async_copy(src_hbm.at[pl.ds(base, _B)], is0, si0)
        cb_ = pltpu.async_copy(dst_hbm.at[pl.ds(base, _B)], id0, si0)
        ca.wait()
        cb_.wait()
        gather_issue(cb0, eb0, is0, id0, sg0)
        gather_wait(cb0, eb0, is0, id0, sg0)
        for t in range(_B // 16):
            sx0[pl.ds(16 * t, 16)] = id0[pl.ds(16 * t, 16)]
        _edge_compute(cb0, eb0, msg0)
        pltpu.sync_copy(msg0, acc.at[sx0], add=True)

    for kk in range(3):
        idx_load(kk, gsets[kk][2], gsets[kk][3], gsets[kk][5])
    idx_wait(0, is0, id0, si0)
    gather_issue(cb0, eb0, is0, id0, sg0)
    idx_wait(1, is1, id1, si1)
    gather_issue(cb1, eb1, is1, id1, sg1)

    def phase(k, G, G2, S):
        (cbG, ebG, isG, idG, sgG, siG) = G
        (cbH, ebH, isH, idH, sgH, siH) = G2
        (msgS, sxS, ssS) = S
        gather_wait(cbG, ebG, isG, idG, sgG)

        @pl.when(k >= 2)
        def _():
            pltpu.make_async_copy(msgS, acc.at[sxS], ssS).wait()

        for t in range(_B // 16):
            sxS[pl.ds(16 * t, 16)] = idG[pl.ds(16 * t, 16)]

        @pl.when(k + 3 <= _CHT - 1)
        def _():
            idx_load(k + 3, isG, idG, siG)

        @pl.when(k + 2 <= _CHT - 1)
        def _():
            idx_wait(k + 2, isH, idH, siH)
            gather_issue(cbH, ebH, isH, idH, sgH)

        _edge_compute(cbG, ebG, msgS)
        pltpu.async_copy(msgS, acc.at[sxS], ssS, add=True)

    def pipe(i, carry):
        for p in range(6):
            phase(6 * i + p, gsets[p % 3], gsets[(p + 2) % 3], ssets[p % 2])
        return carry

    lax.fori_loop(0, _CHT // 6, pipe, 0)
    pltpu.make_async_copy(msg0, acc.at[sx0], ss0).wait()
    pltpu.make_async_copy(msg1, acc.at[sx1], ss1).wait()
    plsc.subcore_barrier()

    for z in range(10):
        r0, rows = (64 * z, 64) if z < 9 else (576, 56)
        r = row0 + r0
        pltpu.sync_copy(acc.at[pl.ds(r, rows)],
                        out_hbm.at[cid, pl.ds(r, rows)])


def _edge_call(src, dst, comb, a_dst):
    mesh = plsc.VectorSubcoreMesh(core_axis_name="c", subcore_axis_name="s",
                                  num_cores=_NC, num_subcores=_NS)
    gset = [
        pltpu.VMEM((_B, 80), jnp.int32),
        pltpu.VMEM((_B, 16), jnp.float32),
        pltpu.VMEM((_B,), jnp.int32),
        pltpu.VMEM((_B,), jnp.int32),
    ]
    sset = [
        pltpu.VMEM((_B, _AW), jnp.float32),
        pltpu.VMEM((_B,), jnp.int32),
    ]
    return pl.kernel(
        _edge_body,
        out_type=jax.ShapeDtypeStruct((_NC, _NP, _AW), jnp.float32),
        mesh=mesh,
        scratch_types=(
            gset * 3 + sset * 2
            + [pltpu.SemaphoreType.DMA] * 8
            + [pltpu.VMEM_SHARED((_NP, _AW), jnp.float32)]
        ),
        compiler_params=pltpu.CompilerParams(use_tc_tiling_on_sc=False,
                                            needs_layout_passes=False),
    )(src, dst, comb, a_dst)


def _build_proj(W, att_s, att_d, perm, tau):
    Wp = W[:, perm]
    rows = jnp.arange(_HH, dtype=jnp.int32)
    cols = rows // _HID
    Bs = jnp.zeros((_HH, _HEADS), jnp.float32).at[rows, cols].set(
        att_s.reshape(-1))[perm]
    Bd = jnp.zeros((_HH, _HEADS), jnp.float32).at[rows, cols].set(
        att_d.reshape(-1))[perm]
    Ps = Wp @ jnp.concatenate([Bs, Bs], axis=1)
    Pd = Wp @ jnp.concatenate([Bd, Bd], axis=1)
    return jnp.concatenate(
        [Wp[:, tau[0::2]], Wp[:, tau[1::2]], Ps, Pd], axis=1)


def kernel(x, edge_index, W1, att_src1, att_dst1, b1,
           W2, att_src2, att_dst2, b2):
    perm = jnp.asarray(_IPERM)
    tau = jnp.asarray(_TAU)
    b1p = b1[perm].reshape(1, _HH)
    WW1 = _build_proj(W1, att_src1, att_dst1, perm, tau)
    WW2 = _build_proj(W2[perm], att_src2, att_dst2, perm, tau)
    src = edge_index[0]
    dst = edge_index[1]
    xp = jnp.pad(x, ((0, _NP - _N), (0, 0)))

    comb1, ad1 = _dense1(xp, WW1)
    acc1 = _edge_call(src, dst, comb1, ad1)
    comb2, ad2 = _dense2(acc1, b1p, WW2)
    acc2 = _edge_call(src, dst, comb2, ad2)
    return _final(acc2, b2.reshape(1, _HID), jnp.asarray(_MEAN))
